# Initial kernel scaffold; baseline (speedup 1.0000x reference)
import jax
import jax.numpy as jnp
from jax.experimental import pallas as pl


def kernel(features, coords, times, Wq, bq, Wk, bk, Wv, bv, Wo, bo, spatial_weight, temporal_weight, gamma_param, W1, b1, W2, b2):
    raise NotImplementedError("write your pallas kernel here")


# speedup vs baseline: 10.7222x; 10.7222x over previous
import jax
import jax.numpy as jnp
from jax.experimental import pallas as pl

_B, _N, _D, _H = 2, 1024, 256, 8
_DH = _D // _H
_KB, _KMIN, _KMAX, _RADIUS = 32, 8, 128, 0.05
_HID = _D // 2
_QBLK = 128
_KC = 128
_INF_BITS = 0x7F800000


def _prep_kernel(feat_ref, coords_ref, times_ref, wq_ref, bq_ref, wk_ref,
                 bk_ref, wv_ref, bv_ref, w1t_ref, scal_ref,
                 qt_out, kt_out, vt_out, at_out, inc_out):
    f32 = jnp.float32
    feats = feat_ref[0]
    cx = coords_ref[0][:, 0:1]
    cy = coords_ref[0][:, 1:2]
    tt = times_ref[0]
    alpha_s = scal_ref[0, 0]
    alpha_t = scal_ref[0, 1]
    gamma = scal_ref[0, 2]

    dx = cx - cx.T
    dy = cy - cy.T
    dt = tt - tt.T
    sp2 = dx * dx + dy * dy
    tp2 = dt * dt

    pdist = jnp.sqrt(sp2 + tp2 + 1e-8)
    ncount = jnp.sum((pdist < _RADIUS).astype(f32), axis=1, keepdims=True)
    density = ncount / (_N * _RADIUS ** 3 + 1e-8)
    mean_density = jnp.mean(density)
    ratio = jnp.exp(gamma * jnp.log(mean_density / (density + 1e-8)))
    kval = jnp.clip(_KB * ratio, _KMIN, _KMAX).astype(jnp.int32)

    wd = jnp.sqrt(alpha_s * sp2 + alpha_t * tp2 + 1e-8)
    rr = jax.lax.broadcasted_iota(jnp.int32, (_N, _N), 0)
    cc = jax.lax.broadcasted_iota(jnp.int32, (_N, _N), 1)
    wd = jnp.where(rr == cc, jnp.inf, wd)

    dbits = jax.lax.bitcast_convert_type(wd, jnp.int32)
    lo0 = jnp.zeros((_N, 1), jnp.int32)
    hi0 = jnp.full((_N, 1), _INF_BITS, jnp.int32)

    def body(_, carry):
        lo, hi = carry
        mid = lo + (hi - lo) // 2
        cnt = jnp.sum((dbits <= mid).astype(jnp.int32), axis=1, keepdims=True)
        pred = cnt >= kval
        return jnp.where(pred, lo, mid + 1), jnp.where(pred, mid, hi)

    _, vbits = jax.lax.fori_loop(0, 31, body, (lo0, hi0))

    less = dbits < vbits
    c_less = jnp.sum(less.astype(jnp.int32), axis=1, keepdims=True)
    eq = dbits == vbits
    eqf = eq.astype(f32)
    upper = (rr < cc).astype(f32)
    tie_rank = jnp.dot(eqf, upper, preferred_element_type=f32)
    room = (kval - c_less).astype(f32)
    include = less | (eq & (tie_rank < room))
    inc_out[0] = include.astype(f32)

    dn = (((0,), (1,)), ((), ()))
    qt_out[0] = jax.lax.dot_general(wq_ref[...], feats, dn,
                                    preferred_element_type=f32) + bq_ref[...]
    kt_out[0] = jax.lax.dot_general(wk_ref[...], feats, dn,
                                    preferred_element_type=f32) + bk_ref[...]
    vt_out[0] = jax.lax.dot_general(wv_ref[...], feats, dn,
                                    preferred_element_type=f32) + bv_ref[...]
    w1t = w1t_ref[...]
    at_out[0] = (w1t[:, 0:1] * cx.T + w1t[:, 1:2] * cy.T + w1t[:, 2:3] * tt.T)


def _attn_kernel(qt_ref, kt_ref, vt_ref, aqt_ref, akt_ref, inc_ref, b1_ref,
                 w2t_ref, b2_ref, wo_ref, bo_ref, out_ref):
    f32 = jnp.float32
    qt = qt_ref[0]
    kt = kt_ref[0]
    vt = vt_ref[0]
    aqt = aqt_ref[0]
    akt = akt_ref[0]
    inc = inc_ref[0]
    b1 = b1_ref[...]
    w2t = w2t_ref[...]
    b2 = b2_ref[...]

    pieces = []
    for c in range(_N // _KC):
        akc = akt[:, c * _KC:(c + 1) * _KC]
        pre = aqt[:, :, None] - akc[:, None, :] + b1[:, :, None]
        g = jax.nn.gelu(pre).reshape(_HID, _QBLK * _KC)
        bc = jnp.dot(w2t, g, preferred_element_type=f32)
        pieces.append(bc.reshape(_H * _QBLK, _KC))
    bias_all = jnp.concatenate(pieces, axis=1)

    scale = _DH ** -0.5
    cdim = (((0,), (0,)), ((), ()))
    vdim = (((1,), (1,)), ((), ()))
    outs = []
    for h in range(_H):
        qh = qt[h * _DH:(h + 1) * _DH, :]
        kh = kt[h * _DH:(h + 1) * _DH, :]
        vh = vt[h * _DH:(h + 1) * _DH, :]
        s = jax.lax.dot_general(qh, kh, cdim,
                                preferred_element_type=f32) * scale
        s = s + bias_all[h * _QBLK:(h + 1) * _QBLK, :] + b2[0, h]
        s = jnp.where(inc > 0.5, s, -1e9)
        m = jnp.max(s, axis=1, keepdims=True)
        e = jnp.exp(s - m)
        attn = e / jnp.sum(e, axis=1, keepdims=True)
        outs.append(jax.lax.dot_general(attn, vh, vdim,
                                        preferred_element_type=f32))
    o = jnp.concatenate(outs, axis=1)
    out_ref[0] = jnp.dot(o, wo_ref[...], preferred_element_type=f32) + bo_ref[...]


def kernel(features, coords, times, Wq, bq, Wk, bk, Wv, bv, Wo, bo,
           spatial_weight, temporal_weight, gamma_param, W1, b1, W2, b2):
    f32 = jnp.float32
    times2 = times[..., None]
    alpha_s = jax.nn.softplus(spatial_weight)
    alpha_t = jax.nn.softplus(temporal_weight)
    gamma = jax.nn.sigmoid(gamma_param)
    scal = jnp.stack([alpha_s, alpha_t, gamma]).reshape(1, 3).astype(f32)

    bq2 = bq.reshape(_D, 1)
    bk2 = bk.reshape(_D, 1)
    bv2 = bv.reshape(_D, 1)
    bo2 = bo.reshape(1, _D)
    b12 = b1.reshape(_HID, 1)
    b22 = b2.reshape(1, _H)
    w1t = W1.T
    w2t = W2.T

    qt, kt, vt, at, inc = pl.pallas_call(
        _prep_kernel,
        grid=(_B,),
        in_specs=[
            pl.BlockSpec((1, _N, _D), lambda b: (b, 0, 0)),
            pl.BlockSpec((1, _N, 2), lambda b: (b, 0, 0)),
            pl.BlockSpec((1, _N, 1), lambda b: (b, 0, 0)),
            pl.BlockSpec((_D, _D), lambda b: (0, 0)),
            pl.BlockSpec((_D, 1), lambda b: (0, 0)),
            pl.BlockSpec((_D, _D), lambda b: (0, 0)),
            pl.BlockSpec((_D, 1), lambda b: (0, 0)),
            pl.BlockSpec((_D, _D), lambda b: (0, 0)),
            pl.BlockSpec((_D, 1), lambda b: (0, 0)),
            pl.BlockSpec((_HID, 3), lambda b: (0, 0)),
            pl.BlockSpec((1, 3), lambda b: (0, 0)),
        ],
        out_specs=[
            pl.BlockSpec((1, _D, _N), lambda b: (b, 0, 0)),
            pl.BlockSpec((1, _D, _N), lambda b: (b, 0, 0)),
            pl.BlockSpec((1, _D, _N), lambda b: (b, 0, 0)),
            pl.BlockSpec((1, _HID, _N), lambda b: (b, 0, 0)),
            pl.BlockSpec((1, _N, _N), lambda b: (b, 0, 0)),
        ],
        out_shape=[
            jax.ShapeDtypeStruct((_B, _D, _N), f32),
            jax.ShapeDtypeStruct((_B, _D, _N), f32),
            jax.ShapeDtypeStruct((_B, _D, _N), f32),
            jax.ShapeDtypeStruct((_B, _HID, _N), f32),
            jax.ShapeDtypeStruct((_B, _N, _N), f32),
        ],
    )(features, coords, times2, Wq, bq2, Wk, bk2, Wv, bv2, w1t, scal)

    nqb = _N // _QBLK
    out = pl.pallas_call(
        _attn_kernel,
        grid=(_B, nqb),
        in_specs=[
            pl.BlockSpec((1, _D, _QBLK), lambda b, i: (b, 0, i)),
            pl.BlockSpec((1, _D, _N), lambda b, i: (b, 0, 0)),
            pl.BlockSpec((1, _D, _N), lambda b, i: (b, 0, 0)),
            pl.BlockSpec((1, _HID, _QBLK), lambda b, i: (b, 0, i)),
            pl.BlockSpec((1, _HID, _N), lambda b, i: (b, 0, 0)),
            pl.BlockSpec((1, _QBLK, _N), lambda b, i: (b, i, 0)),
            pl.BlockSpec((_HID, 1), lambda b, i: (0, 0)),
            pl.BlockSpec((_H, _HID), lambda b, i: (0, 0)),
            pl.BlockSpec((1, _H), lambda b, i: (0, 0)),
            pl.BlockSpec((_D, _D), lambda b, i: (0, 0)),
            pl.BlockSpec((1, _D), lambda b, i: (0, 0)),
        ],
        out_specs=pl.BlockSpec((1, _QBLK, _D), lambda b, i: (b, i, 0)),
        out_shape=jax.ShapeDtypeStruct((_B, _N, _D), f32),
    )(qt, kt, vt, at, at, inc, b12, w2t, b22, Wo, bo2)
    return out


# speedup vs baseline: 17.8636x; 1.6660x over previous
import jax
import jax.numpy as jnp
from jax.experimental import pallas as pl
from jax.experimental.pallas import tpu as pltpu

_B, _N, _D, _H = 2, 1024, 256, 8
_DH = _D // _H
_KB, _KMIN, _KMAX, _RADIUS = 32, 8, 128, 0.05
_HID = _D // 2
_QBLK = 128
_KC = 128
_INF_BITS = 0x7F800000


def _prep_kernel(feat_ref, coords_ref, times_ref, wq_ref, bq_ref, wk_ref,
                 bk_ref, wv_ref, bv_ref, w1t_ref, scal_ref,
                 qt_out, kt_out, vt_out, at_out, inc_out):
    f32 = jnp.float32
    feats = feat_ref[0]
    cx = coords_ref[0][:, 0:1]
    cy = coords_ref[0][:, 1:2]
    tt = times_ref[0]
    alpha_s = scal_ref[0, 0]
    alpha_t = scal_ref[0, 1]
    gamma = scal_ref[0, 2]

    dx = cx - cx.T
    dy = cy - cy.T
    dt = tt - tt.T
    sp2 = dx * dx + dy * dy
    tp2 = dt * dt

    pdist = jnp.sqrt(sp2 + tp2 + 1e-8)
    ncount = jnp.sum((pdist < _RADIUS).astype(f32), axis=1, keepdims=True)
    density = ncount / (_N * _RADIUS ** 3 + 1e-8)
    mean_density = jnp.mean(density)
    ratio = jnp.exp(gamma * jnp.log(mean_density / (density + 1e-8)))
    kval = jnp.clip(_KB * ratio, _KMIN, _KMAX).astype(jnp.int32)

    wd = jnp.sqrt(alpha_s * sp2 + alpha_t * tp2 + 1e-8)
    rr = jax.lax.broadcasted_iota(jnp.int32, (_N, _N), 0)
    cc = jax.lax.broadcasted_iota(jnp.int32, (_N, _N), 1)
    wd = jnp.where(rr == cc, jnp.inf, wd)

    dbits = jax.lax.bitcast_convert_type(wd, jnp.int32)
    lo0 = jnp.zeros((_N, 1), jnp.int32)
    hi0 = jnp.full((_N, 1), _INF_BITS, jnp.int32)

    def body(_, carry):
        lo, hi = carry
        mid = lo + (hi - lo) // 2
        cnt = jnp.sum((dbits <= mid).astype(jnp.int32), axis=1, keepdims=True)
        pred = cnt >= kval
        return jnp.where(pred, lo, mid + 1), jnp.where(pred, mid, hi)

    _, vbits = jax.lax.fori_loop(0, 31, body, (lo0, hi0))

    less = dbits < vbits
    c_less = jnp.sum(less.astype(jnp.int32), axis=1, keepdims=True)
    eq = dbits == vbits
    eqf = eq.astype(f32)
    upper = (rr < cc).astype(f32)
    tie_rank = jnp.dot(eqf, upper, preferred_element_type=f32)
    room = (kval - c_less).astype(f32)
    include = less | (eq & (tie_rank < room))
    inc_out[0] = include.astype(f32)

    dn = (((0,), (1,)), ((), ()))
    qt_out[0] = jax.lax.dot_general(wq_ref[...], feats, dn,
                                    preferred_element_type=f32) + bq_ref[...]
    kt_out[0] = jax.lax.dot_general(wk_ref[...], feats, dn,
                                    preferred_element_type=f32) + bk_ref[...]
    vt_out[0] = jax.lax.dot_general(wv_ref[...], feats, dn,
                                    preferred_element_type=f32) + bv_ref[...]
    w1t = w1t_ref[...]
    at_out[0] = (w1t[:, 0:1] * cx.T + w1t[:, 1:2] * cy.T + w1t[:, 2:3] * tt.T)


def _attn_kernel(qt_ref, kt_ref, vt_ref, aqt_ref, akt_ref, inc_ref, b1_ref,
                 w2t_ref, b2_ref, wo_ref, bo_ref, out_ref):
    f32 = jnp.float32
    qt = qt_ref[0]
    kt = kt_ref[0]
    vt = vt_ref[0]
    aqt = aqt_ref[0]
    akt = akt_ref[0]
    inc = inc_ref[0]
    b1 = b1_ref[...]
    w2t = w2t_ref[...]
    b2 = b2_ref[...]

    bf16 = jnp.bfloat16
    aqt16 = aqt.astype(bf16)
    akt16 = akt.astype(bf16)
    b116 = b1.astype(bf16)
    w2t16 = w2t.astype(bf16)
    pieces = []
    for c in range(_N // _KC):
        akc = akt16[:, c * _KC:(c + 1) * _KC]
        pre = aqt16[:, :, None] - akc[:, None, :] + b116[:, :, None]
        g = jax.nn.gelu(pre).reshape(_HID, _QBLK * _KC)
        bc = jnp.dot(w2t16, g, preferred_element_type=f32)
        pieces.append(bc.reshape(_H * _QBLK, _KC))
    bias_all = jnp.concatenate(pieces, axis=1)

    scale = _DH ** -0.5
    cdim = (((0,), (0,)), ((), ()))
    vdim = (((1,), (1,)), ((), ()))
    outs = []
    for h in range(_H):
        qh = qt[h * _DH:(h + 1) * _DH, :]
        kh = kt[h * _DH:(h + 1) * _DH, :]
        vh = vt[h * _DH:(h + 1) * _DH, :]
        s = jax.lax.dot_general(qh, kh, cdim,
                                preferred_element_type=f32) * scale
        s = s + bias_all[h * _QBLK:(h + 1) * _QBLK, :] + b2[0, h]
        s = jnp.where(inc > 0.5, s, -1e9)
        m = jnp.max(s, axis=1, keepdims=True)
        e = jnp.exp(s - m)
        attn = e / jnp.sum(e, axis=1, keepdims=True)
        outs.append(jax.lax.dot_general(attn, vh, vdim,
                                        preferred_element_type=f32))
    o = jnp.concatenate(outs, axis=1)
    out_ref[0] = jnp.dot(o, wo_ref[...], preferred_element_type=f32) + bo_ref[...]


def kernel(features, coords, times, Wq, bq, Wk, bk, Wv, bv, Wo, bo,
           spatial_weight, temporal_weight, gamma_param, W1, b1, W2, b2):
    f32 = jnp.float32
    times2 = times[..., None]
    alpha_s = jax.nn.softplus(spatial_weight)
    alpha_t = jax.nn.softplus(temporal_weight)
    gamma = jax.nn.sigmoid(gamma_param)
    scal = jnp.stack([alpha_s, alpha_t, gamma]).reshape(1, 3).astype(f32)

    bq2 = bq.reshape(_D, 1)
    bk2 = bk.reshape(_D, 1)
    bv2 = bv.reshape(_D, 1)
    bo2 = bo.reshape(1, _D)
    b12 = b1.reshape(_HID, 1)
    b22 = b2.reshape(1, _H)
    w1t = W1.T
    w2t = W2.T

    qt, kt, vt, at, inc = pl.pallas_call(
        _prep_kernel,
        grid=(_B,),
        compiler_params=pltpu.CompilerParams(
            dimension_semantics=("parallel",)),
        in_specs=[
            pl.BlockSpec((1, _N, _D), lambda b: (b, 0, 0)),
            pl.BlockSpec((1, _N, 2), lambda b: (b, 0, 0)),
            pl.BlockSpec((1, _N, 1), lambda b: (b, 0, 0)),
            pl.BlockSpec((_D, _D), lambda b: (0, 0)),
            pl.BlockSpec((_D, 1), lambda b: (0, 0)),
            pl.BlockSpec((_D, _D), lambda b: (0, 0)),
            pl.BlockSpec((_D, 1), lambda b: (0, 0)),
            pl.BlockSpec((_D, _D), lambda b: (0, 0)),
            pl.BlockSpec((_D, 1), lambda b: (0, 0)),
            pl.BlockSpec((_HID, 3), lambda b: (0, 0)),
            pl.BlockSpec((1, 3), lambda b: (0, 0)),
        ],
        out_specs=[
            pl.BlockSpec((1, _D, _N), lambda b: (b, 0, 0)),
            pl.BlockSpec((1, _D, _N), lambda b: (b, 0, 0)),
            pl.BlockSpec((1, _D, _N), lambda b: (b, 0, 0)),
            pl.BlockSpec((1, _HID, _N), lambda b: (b, 0, 0)),
            pl.BlockSpec((1, _N, _N), lambda b: (b, 0, 0)),
        ],
        out_shape=[
            jax.ShapeDtypeStruct((_B, _D, _N), f32),
            jax.ShapeDtypeStruct((_B, _D, _N), f32),
            jax.ShapeDtypeStruct((_B, _D, _N), f32),
            jax.ShapeDtypeStruct((_B, _HID, _N), f32),
            jax.ShapeDtypeStruct((_B, _N, _N), f32),
        ],
    )(features, coords, times2, Wq, bq2, Wk, bk2, Wv, bv2, w1t, scal)

    nqb = _N // _QBLK
    out = pl.pallas_call(
        _attn_kernel,
        grid=(_B, nqb),
        compiler_params=pltpu.CompilerParams(
            dimension_semantics=("parallel", "parallel")),
        in_specs=[
            pl.BlockSpec((1, _D, _QBLK), lambda b, i: (b, 0, i)),
            pl.BlockSpec((1, _D, _N), lambda b, i: (b, 0, 0)),
            pl.BlockSpec((1, _D, _N), lambda b, i: (b, 0, 0)),
            pl.BlockSpec((1, _HID, _QBLK), lambda b, i: (b, 0, i)),
            pl.BlockSpec((1, _HID, _N), lambda b, i: (b, 0, 0)),
            pl.BlockSpec((1, _QBLK, _N), lambda b, i: (b, i, 0)),
            pl.BlockSpec((_HID, 1), lambda b, i: (0, 0)),
            pl.BlockSpec((_H, _HID), lambda b, i: (0, 0)),
            pl.BlockSpec((1, _H), lambda b, i: (0, 0)),
            pl.BlockSpec((_D, _D), lambda b, i: (0, 0)),
            pl.BlockSpec((1, _D), lambda b, i: (0, 0)),
        ],
        out_specs=pl.BlockSpec((1, _QBLK, _D), lambda b, i: (b, i, 0)),
        out_shape=jax.ShapeDtypeStruct((_B, _N, _D), f32),
    )(qt, kt, vt, at, at, inc, b12, w2t, b22, Wo, bo2)
    return out
